# baseline (speedup 1.0000x reference)
import functools
import math

import jax
import jax.numpy as jnp
from jax import lax
from jax.experimental import pallas as pl
from jax.experimental.pallas import tpu as pltpu
from jax.experimental.pallas import tpu_sc as plsc

B, T, C = 1, 2048, 768
E = 8
TOPK = 2
H = 4 * C
N = B * T
S = N * TOPK

BLK = 512
G = (S - (E - 1) + BLK - 1) // BLK + (E - 1)
NPAD = G * BLK

NW = 32
TOKW = N // NW
LANES = 16
WROW = 128
C2 = C // 2


def _router_body(x_ref, wr_ref, br_ref,
                 aux_ref, w0_ref, w1_ref, pos0_ref, pos1_ref,
                 bexp_ref, nact_ref):
    xb = x_ref[...]
    logits = jnp.dot(xb, wr_ref[...], preferred_element_type=jnp.float32)
    logits = logits + br_ref[...]
    m = jnp.max(logits, axis=1, keepdims=True)
    ex = jnp.exp(logits - m)
    gates = ex / jnp.sum(ex, axis=1, keepdims=True)

    mean_gates = jnp.sum(gates, axis=0, keepdims=True) / float(N)
    aux_ref[...] = jnp.mean((mean_gates - 1.0 / E) ** 2).reshape(1, 1)

    lane = lax.broadcasted_iota(jnp.int32, (N, E), 1)
    p0 = jnp.max(gates, axis=1, keepdims=True)
    e0 = jnp.min(jnp.where(gates == p0, lane, E), axis=1, keepdims=True)
    g2 = jnp.where(lane == e0, -jnp.inf, gates)
    p1 = jnp.max(g2, axis=1, keepdims=True)
    e1 = jnp.min(jnp.where(g2 == p1, lane, E), axis=1, keepdims=True)

    w0_ref[...] = p0 * jnp.ones((1, WROW), jnp.float32)
    w1_ref[...] = p1 * jnp.ones((1, WROW), jnp.float32)

    oh0 = (lane == e0).astype(jnp.float32)
    oh1 = (lane == e1).astype(jnp.float32)
    cat = jnp.concatenate([oh0, oh1], axis=0)

    c3 = cat.reshape(S // 8, 8, E)
    for k in (1, 2, 4):
        sh = jnp.concatenate(
            [jnp.zeros((S // 8, k, E), jnp.float32), c3[:, :8 - k, :]], axis=1)
        c3 = c3 + sh
    chunk_tot = c3[:, 7, :]
    ct = chunk_tot
    for k in (1, 2, 4, 8, 16, 32, 64, 128, 256):
        sh = jnp.concatenate(
            [jnp.zeros((k, E), jnp.float32), ct[:512 - k, :]], axis=0)
        ct = ct + sh
    excl_chunk = ct - chunk_tot
    csum = c3 + excl_chunk[:, None, :]
    csum = csum.reshape(S, E)

    counts = ct[511:512, :]
    pc = jnp.floor((counts + (BLK - 1)) / BLK) * BLK

    starts = []
    cumincl = []
    acc = jnp.zeros((1, 1), jnp.float32)
    for e in range(E):
        pce = lax.slice(pc, (0, e), (1, e + 1))
        starts.append(acc)
        acc = acc + pce
        cumincl.append(acc)
    start_row = jnp.concatenate(starts, axis=1)

    pos_all = jnp.sum(cat * (start_row + csum), axis=1, keepdims=True) - 1.0
    pos_all = pos_all.astype(jnp.int32)
    pos0_ref[...] = pos_all[:N]
    pos1_ref[...] = pos_all[N:]

    bstart = lax.broadcasted_iota(jnp.int32, (G + 1, 1), 0).astype(
        jnp.float32) * BLK
    be = jnp.zeros((G + 1, 1), jnp.float32)
    for e in range(E):
        be = be + (cumincl[e] <= bstart).astype(jnp.float32)
    bexp_ref[...] = jnp.minimum(be, E - 1).astype(jnp.int32)
    nact_ref[...] = (acc / BLK).astype(jnp.int32)


def _run_router(x_flat, Wr, br2):
    return pl.pallas_call(
        _router_body,
        in_specs=[
            pl.BlockSpec((N, C), lambda: (0, 0)),
            pl.BlockSpec((C, E), lambda: (0, 0)),
            pl.BlockSpec((1, E), lambda: (0, 0)),
        ],
        out_specs=[
            pl.BlockSpec((1, 1), lambda: (0, 0)),
            pl.BlockSpec((N, WROW), lambda: (0, 0)),
            pl.BlockSpec((N, WROW), lambda: (0, 0)),
            pl.BlockSpec((N, 1), lambda: (0, 0)),
            pl.BlockSpec((N, 1), lambda: (0, 0)),
            pl.BlockSpec((G + 1, 1), lambda: (0, 0)),
            pl.BlockSpec((1, 1), lambda: (0, 0)),
        ],
        out_shape=[
            jax.ShapeDtypeStruct((1, 1), jnp.float32),
            jax.ShapeDtypeStruct((N, WROW), jnp.float32),
            jax.ShapeDtypeStruct((N, WROW), jnp.float32),
            jax.ShapeDtypeStruct((N, 1), jnp.int32),
            jax.ShapeDtypeStruct((N, 1), jnp.int32),
            jax.ShapeDtypeStruct((G + 1, 1), jnp.int32),
            jax.ShapeDtypeStruct((1, 1), jnp.int32),
        ],
    )(x_flat, Wr, br2)


@functools.cache
def _get_sc_dispatch():
    mesh = plsc.VectorSubcoreMesh(core_axis_name="c", subcore_axis_name="s")

    @functools.partial(
        pl.kernel,
        mesh=mesh,
        out_type=[
            jax.ShapeDtypeStruct((NPAD, C2), jnp.float32),
            jax.ShapeDtypeStruct((NPAD, WROW), jnp.float32),
        ],
        scratch_types=[
            pltpu.VMEM((TOKW,), jnp.int32),
            pltpu.VMEM((TOKW,), jnp.int32),
            pltpu.VMEM((TOKW, C2), jnp.float32),
            pltpu.VMEM((TOKW, WROW), jnp.float32),
            pltpu.VMEM((TOKW, WROW), jnp.float32),
            pltpu.SemaphoreType.DMA,
        ],
    )
    def _sc_dispatch(x_hbm, w0_hbm, w1_hbm, pos0_hbm, pos1_hbm,
                     out_hbm, ws_hbm, idx0_v, idx1_v, rows_v, w0_v, w1_v,
                     sem):
        wid = lax.axis_index("s") * 2 + lax.axis_index("c")
        base = wid * TOKW
        pltpu.sync_copy(pos0_hbm.at[pl.ds(base, TOKW)], idx0_v)
        pltpu.sync_copy(pos1_hbm.at[pl.ds(base, TOKW)], idx1_v)
        pltpu.sync_copy(x_hbm.at[pl.ds(base, TOKW)], rows_v)
        pltpu.sync_copy(w0_hbm.at[pl.ds(base, TOKW)], w0_v)
        pltpu.sync_copy(w1_hbm.at[pl.ds(base, TOKW)], w1_v)
        cpa = pltpu.async_copy(rows_v, out_hbm.at[idx0_v], sem)
        cpb = pltpu.async_copy(rows_v, out_hbm.at[idx1_v], sem)
        cpc = pltpu.async_copy(w0_v, ws_hbm.at[idx0_v], sem)
        cpd = pltpu.async_copy(w1_v, ws_hbm.at[idx1_v], sem)
        cpa.wait()
        cpb.wait()
        cpc.wait()
        cpd.wait()

    return _sc_dispatch


def _ffn_body(bexp_ref, nact_ref, xs_ref, ws_ref, w1_ref, b1_ref, w2_ref,
              b2_ref, o_ref):
    b = pl.program_id(0)

    @pl.when(b < nact_ref[0])
    def _():
        h = jnp.dot(xs_ref[...].astype(jnp.float32), w1_ref[0],
                    preferred_element_type=jnp.float32)
        h = h + b1_ref[0]
        h = 0.5 * h * (1.0 + lax.erf(h * (1.0 / math.sqrt(2.0))))
        y = jnp.dot(h, w2_ref[0], preferred_element_type=jnp.float32)
        o_ref[...] = (y + b2_ref[0]) * ws_ref[:, 0:1]


def _run_ffn(bexp, nact, xs, ws, W1, b1r, W2, b2r):
    grid_spec = pltpu.PrefetchScalarGridSpec(
        num_scalar_prefetch=2,
        grid=(G,),
        in_specs=[
            pl.BlockSpec((BLK, C), lambda b, be, na: (b, 0)),
            pl.BlockSpec((BLK, WROW), lambda b, be, na: (b, 0)),
            pl.BlockSpec((1, C, H), lambda b, be, na: (be[b], 0, 0)),
            pl.BlockSpec((1, 1, H), lambda b, be, na: (be[b], 0, 0)),
            pl.BlockSpec((1, H, C), lambda b, be, na: (be[b], 0, 0)),
            pl.BlockSpec((1, 1, C), lambda b, be, na: (be[b], 0, 0)),
        ],
        out_specs=pl.BlockSpec((BLK, C), lambda b, be, na: (b, 0)),
    )
    return pl.pallas_call(
        _ffn_body,
        grid_spec=grid_spec,
        out_shape=jax.ShapeDtypeStruct((NPAD, C), jnp.float32),
    )(bexp, nact, xs, ws, W1, b1r, W2, b2r)


@functools.cache
def _get_sc_combine():
    mesh = plsc.VectorSubcoreMesh(core_axis_name="c", subcore_axis_name="s")

    @functools.partial(
        pl.kernel,
        mesh=mesh,
        out_type=jax.ShapeDtypeStruct((N, C), jnp.float32),
        scratch_types=[
            pltpu.VMEM((TOKW,), jnp.int32),
            pltpu.VMEM((TOKW,), jnp.int32),
            pltpu.VMEM((TOKW, C), jnp.float32),
            pltpu.VMEM((TOKW, C), jnp.float32),
            pltpu.SemaphoreType.DMA,
        ],
    )
    def _sc_combine(y_hbm, pos0_hbm, pos1_hbm, out_hbm,
                    idx0_v, idx1_v, a_v, b_v, sem):
        wid = lax.axis_index("s") * 2 + lax.axis_index("c")
        base = wid * TOKW
        pltpu.sync_copy(pos0_hbm.at[pl.ds(base, TOKW)], idx0_v)
        pltpu.sync_copy(pos1_hbm.at[pl.ds(base, TOKW)], idx1_v)
        cp0 = pltpu.async_copy(y_hbm.at[idx0_v], a_v, sem)
        cp1 = pltpu.async_copy(y_hbm.at[idx1_v], b_v, sem)
        cp0.wait()
        cp1.wait()

        def body(t, carry):
            for cc in range(C // LANES):
                sl = pl.ds(cc * LANES, LANES)
                a_v[t, sl] = a_v[t, sl] + b_v[t, sl]
            return carry

        lax.fori_loop(0, TOKW, body, 0)
        pltpu.sync_copy(a_v, out_hbm.at[pl.ds(base, TOKW)])

    return _sc_combine


def kernel(x, Wr, br, W1, b1, W2, b2):
    x_flat = x.reshape(N, C)
    br2 = br.reshape(1, E)
    b1r = b1.reshape(E, 1, H)
    b2r = b2.reshape(E, 1, C)

    aux, w0, w1, pos0, pos1, bexp, nact = _run_router(x_flat, Wr, br2)
    pos0f = pos0.reshape(N)
    pos1f = pos1.reshape(N)

    x16 = x_flat.astype(jnp.bfloat16)
    x32 = lax.bitcast_convert_type(x16.reshape(N, C2, 2), jnp.float32)
    xs32, ws = _get_sc_dispatch()(x32, w0, w1, pos0f, pos1f)
    xs16 = lax.bitcast_convert_type(xs32, jnp.bfloat16).reshape(NPAD, C)
    ys = _run_ffn(bexp.reshape(G + 1), nact.reshape(1), xs16, ws,
                  W1, b1r, W2, b2r)
    out = _get_sc_combine()(ys, pos0f, pos1f)

    return out.reshape(B, T, C), aux[0, 0]


# speedup vs baseline: 2.0839x; 2.0839x over previous
import functools
import math

import jax
import jax.numpy as jnp
from jax import lax
from jax.experimental import pallas as pl
from jax.experimental.pallas import tpu as pltpu
from jax.experimental.pallas import tpu_sc as plsc

B, T, C = 1, 2048, 768
E = 8
TOPK = 2
H = 4 * C
N = B * T
S = N * TOPK

BLK = 512
G = (S - (E - 1) + BLK - 1) // BLK + (E - 1)
NPAD = G * BLK

NW = 32
TOKW = N // NW
LANES = 16
WROW = 128
C2 = C // 2


def _router_body(x_ref, wr_ref, br_ref,
                 aux_ref, w0_ref, w1_ref, pos0_ref, pos1_ref,
                 bexp_ref, nact_ref):
    xb = x_ref[...]
    logits = jnp.dot(xb, wr_ref[...], preferred_element_type=jnp.float32)
    logits = logits + br_ref[...]
    m = jnp.max(logits, axis=1, keepdims=True)
    ex = jnp.exp(logits - m)
    gates = ex / jnp.sum(ex, axis=1, keepdims=True)

    mean_gates = jnp.sum(gates, axis=0, keepdims=True) / float(N)
    aux_ref[...] = jnp.mean((mean_gates - 1.0 / E) ** 2).reshape(1, 1)

    lane = lax.broadcasted_iota(jnp.int32, (N, E), 1)
    p0 = jnp.max(gates, axis=1, keepdims=True)
    e0 = jnp.min(jnp.where(gates == p0, lane, E), axis=1, keepdims=True)
    g2 = jnp.where(lane == e0, -jnp.inf, gates)
    p1 = jnp.max(g2, axis=1, keepdims=True)
    e1 = jnp.min(jnp.where(g2 == p1, lane, E), axis=1, keepdims=True)

    w0_ref[...] = p0 * jnp.ones((1, WROW), jnp.float32)
    w1_ref[...] = p1 * jnp.ones((1, WROW), jnp.float32)

    oh0 = (lane == e0).astype(jnp.float32)
    oh1 = (lane == e1).astype(jnp.float32)
    cat = jnp.concatenate([oh0, oh1], axis=0)

    c3 = cat.reshape(S // 8, 8, E)
    for k in (1, 2, 4):
        sh = jnp.concatenate(
            [jnp.zeros((S // 8, k, E), jnp.float32), c3[:, :8 - k, :]], axis=1)
        c3 = c3 + sh
    chunk_tot = c3[:, 7, :]
    ct = chunk_tot
    for k in (1, 2, 4, 8, 16, 32, 64, 128, 256):
        sh = jnp.concatenate(
            [jnp.zeros((k, E), jnp.float32), ct[:512 - k, :]], axis=0)
        ct = ct + sh
    excl_chunk = ct - chunk_tot
    csum = c3 + excl_chunk[:, None, :]
    csum = csum.reshape(S, E)

    counts = ct[511:512, :]
    pc = jnp.floor((counts + (BLK - 1)) / BLK) * BLK

    starts = []
    cumincl = []
    acc = jnp.zeros((1, 1), jnp.float32)
    for e in range(E):
        pce = lax.slice(pc, (0, e), (1, e + 1))
        starts.append(acc)
        acc = acc + pce
        cumincl.append(acc)
    start_row = jnp.concatenate(starts, axis=1)

    pos_all = jnp.sum(cat * (start_row + csum), axis=1, keepdims=True) - 1.0
    pos_all = pos_all.astype(jnp.int32)
    pos0_ref[...] = pos_all[:N]
    pos1_ref[...] = pos_all[N:]

    bstart = lax.broadcasted_iota(jnp.int32, (G + 1, 1), 0).astype(
        jnp.float32) * BLK
    be = jnp.zeros((G + 1, 1), jnp.float32)
    for e in range(E):
        be = be + (cumincl[e] <= bstart).astype(jnp.float32)
    bexp_ref[...] = jnp.minimum(be, E - 1).astype(jnp.int32)
    nact_ref[...] = (acc / BLK).astype(jnp.int32)


def _run_router(x_flat, Wr, br2):
    return pl.pallas_call(
        _router_body,
        in_specs=[
            pl.BlockSpec((N, C), lambda: (0, 0)),
            pl.BlockSpec((C, E), lambda: (0, 0)),
            pl.BlockSpec((1, E), lambda: (0, 0)),
        ],
        out_specs=[
            pl.BlockSpec((1, 1), lambda: (0, 0)),
            pl.BlockSpec((N, WROW), lambda: (0, 0)),
            pl.BlockSpec((N, WROW), lambda: (0, 0)),
            pl.BlockSpec((N, 1), lambda: (0, 0)),
            pl.BlockSpec((N, 1), lambda: (0, 0)),
            pl.BlockSpec((G + 1, 1), lambda: (0, 0)),
            pl.BlockSpec((1, 1), lambda: (0, 0)),
        ],
        out_shape=[
            jax.ShapeDtypeStruct((1, 1), jnp.float32),
            jax.ShapeDtypeStruct((N, WROW), jnp.float32),
            jax.ShapeDtypeStruct((N, WROW), jnp.float32),
            jax.ShapeDtypeStruct((N, 1), jnp.int32),
            jax.ShapeDtypeStruct((N, 1), jnp.int32),
            jax.ShapeDtypeStruct((G + 1, 1), jnp.int32),
            jax.ShapeDtypeStruct((1, 1), jnp.int32),
        ],
    )(x_flat, Wr, br2)


@functools.cache
def _get_sc_dispatch():
    mesh = plsc.VectorSubcoreMesh(core_axis_name="c", subcore_axis_name="s")

    @functools.partial(
        pl.kernel,
        mesh=mesh,
        out_type=[
            jax.ShapeDtypeStruct((NPAD, C), jnp.float32),
            jax.ShapeDtypeStruct((NPAD, WROW), jnp.float32),
        ],
        scratch_types=[
            pltpu.VMEM((TOKW,), jnp.int32),
            pltpu.VMEM((TOKW,), jnp.int32),
            pltpu.VMEM((TOKW, C), jnp.float32),
            pltpu.VMEM((TOKW, WROW), jnp.float32),
            pltpu.VMEM((TOKW, WROW), jnp.float32),
            pltpu.SemaphoreType.DMA,
        ],
    )
    def _sc_dispatch(x_hbm, w0_hbm, w1_hbm, pos0_hbm, pos1_hbm,
                     out_hbm, ws_hbm, idx0_v, idx1_v, rows_v, w0_v, w1_v,
                     sem):
        wid = lax.axis_index("s") * 2 + lax.axis_index("c")
        base = wid * TOKW
        pltpu.sync_copy(pos0_hbm.at[pl.ds(base, TOKW)], idx0_v)
        pltpu.sync_copy(pos1_hbm.at[pl.ds(base, TOKW)], idx1_v)
        pltpu.sync_copy(x_hbm.at[pl.ds(base, TOKW)], rows_v)
        pltpu.sync_copy(w0_hbm.at[pl.ds(base, TOKW)], w0_v)
        pltpu.sync_copy(w1_hbm.at[pl.ds(base, TOKW)], w1_v)
        cpa = pltpu.async_copy(rows_v, out_hbm.at[idx0_v], sem)
        cpb = pltpu.async_copy(rows_v, out_hbm.at[idx1_v], sem)
        cpc = pltpu.async_copy(w0_v, ws_hbm.at[idx0_v], sem)
        cpd = pltpu.async_copy(w1_v, ws_hbm.at[idx1_v], sem)
        cpa.wait()
        cpb.wait()
        cpc.wait()
        cpd.wait()

    return _sc_dispatch


def _ffn_body(bexp_ref, nact_ref, xs_ref, ws_ref, w1_ref, b1_ref, w2_ref,
              b2_ref, o_ref):
    b = pl.program_id(0)

    @pl.when(b < nact_ref[0])
    def _():
        h = jnp.dot(xs_ref[...], w1_ref[0], preferred_element_type=jnp.float32)
        h = h + b1_ref[0]
        h = 0.5 * h * (1.0 + lax.erf(h * (1.0 / math.sqrt(2.0))))
        y = jnp.dot(h, w2_ref[0], preferred_element_type=jnp.float32)
        o_ref[...] = (y + b2_ref[0]) * ws_ref[:, 0:1]


def _run_ffn(bexp, nact, xs, ws, W1, b1r, W2, b2r):
    grid_spec = pltpu.PrefetchScalarGridSpec(
        num_scalar_prefetch=2,
        grid=(G,),
        in_specs=[
            pl.BlockSpec((BLK, C), lambda b, be, na: (b, 0)),
            pl.BlockSpec((BLK, WROW), lambda b, be, na: (b, 0)),
            pl.BlockSpec((1, C, H), lambda b, be, na: (be[b], 0, 0)),
            pl.BlockSpec((1, 1, H), lambda b, be, na: (be[b], 0, 0)),
            pl.BlockSpec((1, H, C), lambda b, be, na: (be[b], 0, 0)),
            pl.BlockSpec((1, 1, C), lambda b, be, na: (be[b], 0, 0)),
        ],
        out_specs=pl.BlockSpec((BLK, C), lambda b, be, na: (b, 0)),
    )
    return pl.pallas_call(
        _ffn_body,
        grid_spec=grid_spec,
        out_shape=jax.ShapeDtypeStruct((NPAD, C), jnp.float32),
    )(bexp, nact, xs, ws, W1, b1r, W2, b2r)


@functools.cache
def _get_sc_combine():
    mesh = plsc.VectorSubcoreMesh(core_axis_name="c", subcore_axis_name="s")

    @functools.partial(
        pl.kernel,
        mesh=mesh,
        out_type=jax.ShapeDtypeStruct((N, C), jnp.float32),
        scratch_types=[
            pltpu.VMEM((TOKW,), jnp.int32),
            pltpu.VMEM((TOKW,), jnp.int32),
            pltpu.VMEM((TOKW, C), jnp.float32),
            pltpu.VMEM((TOKW, C), jnp.float32),
            pltpu.SemaphoreType.DMA,
        ],
    )
    def _sc_combine(y_hbm, pos0_hbm, pos1_hbm, out_hbm,
                    idx0_v, idx1_v, a_v, b_v, sem):
        wid = lax.axis_index("s") * 2 + lax.axis_index("c")
        base = wid * TOKW
        pltpu.sync_copy(pos0_hbm.at[pl.ds(base, TOKW)], idx0_v)
        pltpu.sync_copy(pos1_hbm.at[pl.ds(base, TOKW)], idx1_v)
        cp0 = pltpu.async_copy(y_hbm.at[idx0_v], a_v, sem)
        cp1 = pltpu.async_copy(y_hbm.at[idx1_v], b_v, sem)
        cp0.wait()
        cp1.wait()

        def body(t, carry):
            for cc in range(C // LANES):
                sl = pl.ds(cc * LANES, LANES)
                a_v[t, sl] = a_v[t, sl] + b_v[t, sl]
            return carry

        lax.fori_loop(0, TOKW, body, 0)
        pltpu.sync_copy(a_v, out_hbm.at[pl.ds(base, TOKW)])

    return _sc_combine


def kernel(x, Wr, br, W1, b1, W2, b2):
    x_flat = x.reshape(N, C)
    br2 = br.reshape(1, E)
    b1r = b1.reshape(E, 1, H)
    b2r = b2.reshape(E, 1, C)

    aux, w0, w1, pos0, pos1, bexp, nact = _run_router(x_flat, Wr, br2)
    pos0f = pos0.reshape(N)
    pos1f = pos1.reshape(N)

    xs, ws = _get_sc_dispatch()(x_flat, w0, w1, pos0f, pos1f)
    ys = _run_ffn(bexp.reshape(G + 1), nact.reshape(1), xs, ws,
                  W1, b1r, W2, b2r)
    out = _get_sc_combine()(ys, pos0f, pos1f)

    return out.reshape(B, T, C), aux[0, 0]


# speedup vs baseline: 2.1220x; 1.0183x over previous
import functools
import math

import jax
import jax.numpy as jnp
from jax import lax
from jax.experimental import pallas as pl
from jax.experimental.pallas import tpu as pltpu
from jax.experimental.pallas import tpu_sc as plsc

B, T, C = 1, 2048, 768
E = 8
TOPK = 2
H = 4 * C
N = B * T
S = N * TOPK

BLK = 512
G = (S - (E - 1) + BLK - 1) // BLK + (E - 1)
NPAD = G * BLK

NW = 32
TOKW = N // NW
LANES = 16
WROW = 128
C2 = C // 2


def _router_body(x_ref, wr_ref, br_ref,
                 aux_ref, w0_ref, w1_ref, pos0_ref, pos1_ref,
                 bexp_ref, nact_ref):
    xb = x_ref[...]
    logits = jnp.dot(xb, wr_ref[...], preferred_element_type=jnp.float32)
    logits = logits + br_ref[...]
    m = jnp.max(logits, axis=1, keepdims=True)
    ex = jnp.exp(logits - m)
    gates = ex / jnp.sum(ex, axis=1, keepdims=True)

    mean_gates = jnp.sum(gates, axis=0, keepdims=True) / float(N)
    aux_ref[...] = jnp.mean((mean_gates - 1.0 / E) ** 2).reshape(1, 1)

    lane = lax.broadcasted_iota(jnp.int32, (N, E), 1)
    p0 = jnp.max(gates, axis=1, keepdims=True)
    e0 = jnp.min(jnp.where(gates == p0, lane, E), axis=1, keepdims=True)
    g2 = jnp.where(lane == e0, -jnp.inf, gates)
    p1 = jnp.max(g2, axis=1, keepdims=True)
    e1 = jnp.min(jnp.where(g2 == p1, lane, E), axis=1, keepdims=True)

    w0_ref[...] = p0 * jnp.ones((1, WROW), jnp.float32)
    w1_ref[...] = p1 * jnp.ones((1, WROW), jnp.float32)

    oh0 = (lane == e0).astype(jnp.float32)
    oh1 = (lane == e1).astype(jnp.float32)
    cat = jnp.concatenate([oh0, oh1], axis=0)

    c3 = cat.reshape(S // 8, 8, E)
    for k in (1, 2, 4):
        sh = jnp.concatenate(
            [jnp.zeros((S // 8, k, E), jnp.float32), c3[:, :8 - k, :]], axis=1)
        c3 = c3 + sh
    chunk_tot = c3[:, 7, :]
    ct = chunk_tot
    for k in (1, 2, 4, 8, 16, 32, 64, 128, 256):
        sh = jnp.concatenate(
            [jnp.zeros((k, E), jnp.float32), ct[:512 - k, :]], axis=0)
        ct = ct + sh
    excl_chunk = ct - chunk_tot
    csum = c3 + excl_chunk[:, None, :]
    csum = csum.reshape(S, E)

    counts = ct[511:512, :]
    pc = jnp.floor((counts + (BLK - 1)) / BLK) * BLK

    starts = []
    cumincl = []
    acc = jnp.zeros((1, 1), jnp.float32)
    for e in range(E):
        pce = lax.slice(pc, (0, e), (1, e + 1))
        starts.append(acc)
        acc = acc + pce
        cumincl.append(acc)
    start_row = jnp.concatenate(starts, axis=1)

    pos_all = jnp.sum(cat * (start_row + csum), axis=1, keepdims=True) - 1.0
    pos_all = pos_all.astype(jnp.int32)
    pos0_ref[...] = pos_all[:N]
    pos1_ref[...] = pos_all[N:]

    bstart = lax.broadcasted_iota(jnp.int32, (G + 1, 1), 0).astype(
        jnp.float32) * BLK
    be = jnp.zeros((G + 1, 1), jnp.float32)
    for e in range(E):
        be = be + (cumincl[e] <= bstart).astype(jnp.float32)
    bexp_ref[...] = jnp.minimum(be, E - 1).astype(jnp.int32)
    nact_ref[...] = (acc / BLK).astype(jnp.int32)


def _run_router(x_flat, Wr, br2):
    return pl.pallas_call(
        _router_body,
        in_specs=[
            pl.BlockSpec((N, C), lambda: (0, 0)),
            pl.BlockSpec((C, E), lambda: (0, 0)),
            pl.BlockSpec((1, E), lambda: (0, 0)),
        ],
        out_specs=[
            pl.BlockSpec((1, 1), lambda: (0, 0)),
            pl.BlockSpec((N, WROW), lambda: (0, 0)),
            pl.BlockSpec((N, WROW), lambda: (0, 0)),
            pl.BlockSpec((N, 1), lambda: (0, 0)),
            pl.BlockSpec((N, 1), lambda: (0, 0)),
            pl.BlockSpec((G + 1, 1), lambda: (0, 0)),
            pl.BlockSpec((1, 1), lambda: (0, 0)),
        ],
        out_shape=[
            jax.ShapeDtypeStruct((1, 1), jnp.float32),
            jax.ShapeDtypeStruct((N, WROW), jnp.float32),
            jax.ShapeDtypeStruct((N, WROW), jnp.float32),
            jax.ShapeDtypeStruct((N, 1), jnp.int32),
            jax.ShapeDtypeStruct((N, 1), jnp.int32),
            jax.ShapeDtypeStruct((G + 1, 1), jnp.int32),
            jax.ShapeDtypeStruct((1, 1), jnp.int32),
        ],
    )(x_flat, Wr, br2)


@functools.cache
def _get_sc_dispatch():
    mesh = plsc.VectorSubcoreMesh(core_axis_name="c", subcore_axis_name="s")

    @functools.partial(
        pl.kernel,
        mesh=mesh,
        out_type=[
            jax.ShapeDtypeStruct((NPAD, C), jnp.float32),
            jax.ShapeDtypeStruct((NPAD, WROW), jnp.float32),
        ],
        scratch_types=[
            pltpu.VMEM((TOKW,), jnp.int32),
            pltpu.VMEM((TOKW,), jnp.int32),
            pltpu.VMEM((TOKW, C), jnp.float32),
            pltpu.VMEM((TOKW, WROW), jnp.float32),
            pltpu.VMEM((TOKW, WROW), jnp.float32),
            pltpu.SemaphoreType.DMA,
            pltpu.SemaphoreType.DMA,
        ],
    )
    def _sc_dispatch(x_hbm, w0_hbm, w1_hbm, pos0_hbm, pos1_hbm,
                     out_hbm, ws_hbm, idx0_v, idx1_v, rows_v, w0_v, w1_v,
                     sem, sem2):
        wid = lax.axis_index("s") * 2 + lax.axis_index("c")
        base = wid * TOKW
        ld0 = pltpu.async_copy(pos0_hbm.at[pl.ds(base, TOKW)], idx0_v, sem2)
        ld1 = pltpu.async_copy(pos1_hbm.at[pl.ds(base, TOKW)], idx1_v, sem2)
        ld2 = pltpu.async_copy(x_hbm.at[pl.ds(base, TOKW)], rows_v, sem2)
        ld3 = pltpu.async_copy(w0_hbm.at[pl.ds(base, TOKW)], w0_v, sem2)
        ld4 = pltpu.async_copy(w1_hbm.at[pl.ds(base, TOKW)], w1_v, sem2)
        ld0.wait()
        ld1.wait()
        ld2.wait()
        ld3.wait()
        ld4.wait()
        cpa = pltpu.async_copy(rows_v, out_hbm.at[idx0_v], sem)
        cpb = pltpu.async_copy(rows_v, out_hbm.at[idx1_v], sem)
        cpc = pltpu.async_copy(w0_v, ws_hbm.at[idx0_v], sem)
        cpd = pltpu.async_copy(w1_v, ws_hbm.at[idx1_v], sem)
        cpa.wait()
        cpb.wait()
        cpc.wait()
        cpd.wait()

    return _sc_dispatch


def _ffn_body(bexp_ref, nact_ref, xs_ref, ws_ref, w1_ref, b1_ref, w2_ref,
              b2_ref, o_ref):
    b = pl.program_id(0)

    @pl.when(b < nact_ref[0])
    def _():
        h = jnp.dot(xs_ref[...], w1_ref[0], preferred_element_type=jnp.float32)
        h = h + b1_ref[0]
        h = 0.5 * h * (1.0 + lax.erf(h * (1.0 / math.sqrt(2.0))))
        y = jnp.dot(h, w2_ref[0], preferred_element_type=jnp.float32)
        o_ref[...] = (y + b2_ref[0]) * ws_ref[:, 0:1]


def _run_ffn(bexp, nact, xs, ws, W1, b1r, W2, b2r):
    grid_spec = pltpu.PrefetchScalarGridSpec(
        num_scalar_prefetch=2,
        grid=(G,),
        in_specs=[
            pl.BlockSpec((BLK, C), lambda b, be, na: (b, 0)),
            pl.BlockSpec((BLK, WROW), lambda b, be, na: (b, 0)),
            pl.BlockSpec((1, C, H), lambda b, be, na: (be[b], 0, 0)),
            pl.BlockSpec((1, 1, H), lambda b, be, na: (be[b], 0, 0)),
            pl.BlockSpec((1, H, C), lambda b, be, na: (be[b], 0, 0)),
            pl.BlockSpec((1, 1, C), lambda b, be, na: (be[b], 0, 0)),
        ],
        out_specs=pl.BlockSpec((BLK, C), lambda b, be, na: (b, 0)),
    )
    return pl.pallas_call(
        _ffn_body,
        grid_spec=grid_spec,
        out_shape=jax.ShapeDtypeStruct((NPAD, C), jnp.float32),
    )(bexp, nact, xs, ws, W1, b1r, W2, b2r)


@functools.cache
def _get_sc_combine():
    mesh = plsc.VectorSubcoreMesh(core_axis_name="c", subcore_axis_name="s")

    @functools.partial(
        pl.kernel,
        mesh=mesh,
        out_type=jax.ShapeDtypeStruct((N, C), jnp.float32),
        scratch_types=[
            pltpu.VMEM((TOKW,), jnp.int32),
            pltpu.VMEM((TOKW,), jnp.int32),
            pltpu.VMEM((TOKW, C), jnp.float32),
            pltpu.VMEM((TOKW, C), jnp.float32),
            pltpu.SemaphoreType.DMA,
            pltpu.SemaphoreType.DMA,
        ],
    )
    def _sc_combine(y_hbm, pos0_hbm, pos1_hbm, out_hbm,
                    idx0_v, idx1_v, a_v, b_v, sem, sem2):
        wid = lax.axis_index("s") * 2 + lax.axis_index("c")
        base = wid * TOKW
        ld0 = pltpu.async_copy(pos0_hbm.at[pl.ds(base, TOKW)], idx0_v, sem2)
        ld1 = pltpu.async_copy(pos1_hbm.at[pl.ds(base, TOKW)], idx1_v, sem2)
        ld0.wait()
        ld1.wait()
        cp0 = pltpu.async_copy(y_hbm.at[idx0_v], a_v, sem)
        cp1 = pltpu.async_copy(y_hbm.at[idx1_v], b_v, sem)
        cp0.wait()
        cp1.wait()

        def body(t, carry):
            for cc in range(C // LANES):
                sl = pl.ds(cc * LANES, LANES)
                a_v[t, sl] = a_v[t, sl] + b_v[t, sl]
            return carry

        lax.fori_loop(0, TOKW, body, 0)
        pltpu.sync_copy(a_v, out_hbm.at[pl.ds(base, TOKW)])

    return _sc_combine


def kernel(x, Wr, br, W1, b1, W2, b2):
    x_flat = x.reshape(N, C)
    br2 = br.reshape(1, E)
    b1r = b1.reshape(E, 1, H)
    b2r = b2.reshape(E, 1, C)

    aux, w0, w1, pos0, pos1, bexp, nact = _run_router(x_flat, Wr, br2)
    pos0f = pos0.reshape(N)
    pos1f = pos1.reshape(N)

    xs, ws = _get_sc_dispatch()(x_flat, w0, w1, pos0f, pos1f)
    ys = _run_ffn(bexp.reshape(G + 1), nact.reshape(1), xs, ws,
                  W1, b1r, W2, b2r)
    out = _get_sc_combine()(ys, pos0f, pos1f)

    return out.reshape(B, T, C), aux[0, 0]


# speedup vs baseline: 2.1450x; 1.0108x over previous
import functools
import math

import jax
import jax.numpy as jnp
from jax import lax
from jax.experimental import pallas as pl
from jax.experimental.pallas import tpu as pltpu
from jax.experimental.pallas import tpu_sc as plsc

B, T, C = 1, 2048, 768
E = 8
TOPK = 2
H = 4 * C
N = B * T
S = N * TOPK

BLK = 512
G = (S - (E - 1) + BLK - 1) // BLK + (E - 1)
NPAD = G * BLK

NW = 32
TOKW = N // NW
LANES = 16
WROW = 128
C2 = C // 2


def _router_body(x_ref, wr_ref, br_ref,
                 aux_ref, w0_ref, w1_ref, pos0_ref, pos1_ref,
                 bexp_ref, nact_ref, ru_ref):
    xb = x_ref[...]
    logits = jnp.dot(xb, wr_ref[...], preferred_element_type=jnp.float32)
    logits = logits + br_ref[...]
    m = jnp.max(logits, axis=1, keepdims=True)
    ex = jnp.exp(logits - m)
    gates = ex / jnp.sum(ex, axis=1, keepdims=True)

    mean_gates = jnp.sum(gates, axis=0, keepdims=True) / float(N)
    aux_ref[...] = jnp.mean((mean_gates - 1.0 / E) ** 2).reshape(1, 1)

    lane = lax.broadcasted_iota(jnp.int32, (N, E), 1)
    p0 = jnp.max(gates, axis=1, keepdims=True)
    e0 = jnp.min(jnp.where(gates == p0, lane, E), axis=1, keepdims=True)
    g2 = jnp.where(lane == e0, -jnp.inf, gates)
    p1 = jnp.max(g2, axis=1, keepdims=True)
    e1 = jnp.min(jnp.where(g2 == p1, lane, E), axis=1, keepdims=True)

    w0_ref[...] = p0 * jnp.ones((1, WROW), jnp.float32)
    w1_ref[...] = p1 * jnp.ones((1, WROW), jnp.float32)

    oh0 = (lane == e0).astype(jnp.float32)
    oh1 = (lane == e1).astype(jnp.float32)
    cat = jnp.concatenate([oh0, oh1], axis=0)

    c3 = cat.reshape(S // 8, 8, E)
    for k in (1, 2, 4):
        sh = jnp.concatenate(
            [jnp.zeros((S // 8, k, E), jnp.float32), c3[:, :8 - k, :]], axis=1)
        c3 = c3 + sh
    chunk_tot = c3[:, 7, :]
    ct = chunk_tot
    for k in (1, 2, 4, 8, 16, 32, 64, 128, 256):
        sh = jnp.concatenate(
            [jnp.zeros((k, E), jnp.float32), ct[:512 - k, :]], axis=0)
        ct = ct + sh
    excl_chunk = ct - chunk_tot
    csum = c3 + excl_chunk[:, None, :]
    csum = csum.reshape(S, E)

    counts = ct[511:512, :]
    pc = jnp.floor((counts + (BLK - 1)) / BLK) * BLK

    starts = []
    cumincl = []
    acc = jnp.zeros((1, 1), jnp.float32)
    for e in range(E):
        pce = lax.slice(pc, (0, e), (1, e + 1))
        starts.append(acc)
        acc = acc + pce
        cumincl.append(acc)
    start_row = jnp.concatenate(starts, axis=1)

    pos_all = jnp.sum(cat * (start_row + csum), axis=1, keepdims=True) - 1.0
    pos_all = pos_all.astype(jnp.int32)
    pos0_ref[...] = pos_all[:N]
    pos1_ref[...] = pos_all[N:]

    bstart = lax.broadcasted_iota(jnp.int32, (G + 1, 1), 0).astype(
        jnp.float32) * BLK
    be = jnp.zeros((G + 1, 1), jnp.float32)
    for e in range(E):
        be = be + (cumincl[e] <= bstart).astype(jnp.float32)
    be = jnp.minimum(be, E - 1)
    bexp_ref[...] = be.astype(jnp.int32)
    nact_ref[...] = (acc / BLK).astype(jnp.int32)

    ren = jnp.zeros((G + 1, 1), jnp.float32)
    for e in range(E):
        cnt_e = lax.slice(counts, (0, e), (1, e + 1))
        ren = ren + jnp.where(be == e, starts[e] + cnt_e, 0.0)
    ru_ref[...] = jnp.clip(ren - bstart, 0.0, BLK).astype(jnp.int32)


def _run_router(x_flat, Wr, br2):
    return pl.pallas_call(
        _router_body,
        in_specs=[
            pl.BlockSpec((N, C), lambda: (0, 0)),
            pl.BlockSpec((C, E), lambda: (0, 0)),
            pl.BlockSpec((1, E), lambda: (0, 0)),
        ],
        out_specs=[
            pl.BlockSpec((1, 1), lambda: (0, 0)),
            pl.BlockSpec((N, WROW), lambda: (0, 0)),
            pl.BlockSpec((N, WROW), lambda: (0, 0)),
            pl.BlockSpec((N, 1), lambda: (0, 0)),
            pl.BlockSpec((N, 1), lambda: (0, 0)),
            pl.BlockSpec((G + 1, 1), lambda: (0, 0)),
            pl.BlockSpec((1, 1), lambda: (0, 0)),
            pl.BlockSpec((G + 1, 1), lambda: (0, 0)),
        ],
        out_shape=[
            jax.ShapeDtypeStruct((1, 1), jnp.float32),
            jax.ShapeDtypeStruct((N, WROW), jnp.float32),
            jax.ShapeDtypeStruct((N, WROW), jnp.float32),
            jax.ShapeDtypeStruct((N, 1), jnp.int32),
            jax.ShapeDtypeStruct((N, 1), jnp.int32),
            jax.ShapeDtypeStruct((G + 1, 1), jnp.int32),
            jax.ShapeDtypeStruct((1, 1), jnp.int32),
            jax.ShapeDtypeStruct((G + 1, 1), jnp.int32),
        ],
    )(x_flat, Wr, br2)


@functools.cache
def _get_sc_dispatch():
    mesh = plsc.VectorSubcoreMesh(core_axis_name="c", subcore_axis_name="s")

    @functools.partial(
        pl.kernel,
        mesh=mesh,
        out_type=[
            jax.ShapeDtypeStruct((NPAD, C), jnp.float32),
            jax.ShapeDtypeStruct((NPAD, WROW), jnp.float32),
        ],
        scratch_types=[
            pltpu.VMEM((TOKW,), jnp.int32),
            pltpu.VMEM((TOKW,), jnp.int32),
            pltpu.VMEM((TOKW, C), jnp.float32),
            pltpu.VMEM((TOKW, WROW), jnp.float32),
            pltpu.VMEM((TOKW, WROW), jnp.float32),
            pltpu.SemaphoreType.DMA,
            pltpu.SemaphoreType.DMA,
        ],
    )
    def _sc_dispatch(x_hbm, w0_hbm, w1_hbm, pos0_hbm, pos1_hbm,
                     out_hbm, ws_hbm, idx0_v, idx1_v, rows_v, w0_v, w1_v,
                     sem, sem2):
        wid = lax.axis_index("s") * 2 + lax.axis_index("c")
        base = wid * TOKW
        ld0 = pltpu.async_copy(pos0_hbm.at[pl.ds(base, TOKW)], idx0_v, sem2)
        ld1 = pltpu.async_copy(pos1_hbm.at[pl.ds(base, TOKW)], idx1_v, sem2)
        ld2 = pltpu.async_copy(x_hbm.at[pl.ds(base, TOKW)], rows_v, sem2)
        ld3 = pltpu.async_copy(w0_hbm.at[pl.ds(base, TOKW)], w0_v, sem2)
        ld4 = pltpu.async_copy(w1_hbm.at[pl.ds(base, TOKW)], w1_v, sem2)
        ld0.wait()
        ld1.wait()
        ld2.wait()
        ld3.wait()
        ld4.wait()
        cpa = pltpu.async_copy(rows_v, out_hbm.at[idx0_v], sem)
        cpb = pltpu.async_copy(rows_v, out_hbm.at[idx1_v], sem)
        cpc = pltpu.async_copy(w0_v, ws_hbm.at[idx0_v], sem)
        cpd = pltpu.async_copy(w1_v, ws_hbm.at[idx1_v], sem)
        cpa.wait()
        cpb.wait()
        cpc.wait()
        cpd.wait()

    return _sc_dispatch


HALF = BLK // 2


def _ffn_body(bexp_ref, nact_ref, ru_ref, xs_ref, ws_ref, w1_ref, b1_ref,
              w2_ref, b2_ref, o_ref):
    b = pl.program_id(0)

    @pl.when(b < nact_ref[0])
    def _():
        full = ru_ref[b] > HALF

        @pl.when(full)
        def _():
            h = jnp.dot(xs_ref[...], w1_ref[0],
                        preferred_element_type=jnp.float32)
            h = h + b1_ref[0]
            h = 0.5 * h * (1.0 + lax.erf(h * (1.0 / math.sqrt(2.0))))
            y = jnp.dot(h, w2_ref[0], preferred_element_type=jnp.float32)
            o_ref[...] = (y + b2_ref[0]) * ws_ref[:, 0:1]

        @pl.when(jnp.logical_not(full))
        def _():
            h = jnp.dot(xs_ref[:HALF, :], w1_ref[0],
                        preferred_element_type=jnp.float32)
            h = h + b1_ref[0]
            h = 0.5 * h * (1.0 + lax.erf(h * (1.0 / math.sqrt(2.0))))
            y = jnp.dot(h, w2_ref[0], preferred_element_type=jnp.float32)
            o_ref[:HALF, :] = (y + b2_ref[0]) * ws_ref[:HALF, 0:1]


def _run_ffn(bexp, nact, ru, xs, ws, W1, b1r, W2, b2r):
    grid_spec = pltpu.PrefetchScalarGridSpec(
        num_scalar_prefetch=3,
        grid=(G,),
        in_specs=[
            pl.BlockSpec((BLK, C), lambda b, be, na, ru: (b, 0)),
            pl.BlockSpec((BLK, WROW), lambda b, be, na, ru: (b, 0)),
            pl.BlockSpec((1, C, H), lambda b, be, na, ru: (be[b], 0, 0)),
            pl.BlockSpec((1, 1, H), lambda b, be, na, ru: (be[b], 0, 0)),
            pl.BlockSpec((1, H, C), lambda b, be, na, ru: (be[b], 0, 0)),
            pl.BlockSpec((1, 1, C), lambda b, be, na, ru: (be[b], 0, 0)),
        ],
        out_specs=pl.BlockSpec((BLK, C), lambda b, be, na, ru: (b, 0)),
    )
    return pl.pallas_call(
        _ffn_body,
        grid_spec=grid_spec,
        out_shape=jax.ShapeDtypeStruct((NPAD, C), jnp.float32),
    )(bexp, nact, ru, xs, ws, W1, b1r, W2, b2r)


@functools.cache
def _get_sc_combine():
    mesh = plsc.VectorSubcoreMesh(core_axis_name="c", subcore_axis_name="s")

    @functools.partial(
        pl.kernel,
        mesh=mesh,
        out_type=jax.ShapeDtypeStruct((N, C), jnp.float32),
        scratch_types=[
            pltpu.VMEM((TOKW,), jnp.int32),
            pltpu.VMEM((TOKW,), jnp.int32),
            pltpu.VMEM((TOKW, C), jnp.float32),
            pltpu.VMEM((TOKW, C), jnp.float32),
            pltpu.SemaphoreType.DMA,
            pltpu.SemaphoreType.DMA,
        ],
    )
    def _sc_combine(y_hbm, pos0_hbm, pos1_hbm, out_hbm,
                    idx0_v, idx1_v, a_v, b_v, sem, sem2):
        wid = lax.axis_index("s") * 2 + lax.axis_index("c")
        base = wid * TOKW
        ld0 = pltpu.async_copy(pos0_hbm.at[pl.ds(base, TOKW)], idx0_v, sem2)
        ld1 = pltpu.async_copy(pos1_hbm.at[pl.ds(base, TOKW)], idx1_v, sem2)
        ld0.wait()
        ld1.wait()
        cp0 = pltpu.async_copy(y_hbm.at[idx0_v], a_v, sem)
        cp1 = pltpu.async_copy(y_hbm.at[idx1_v], b_v, sem)
        cp0.wait()
        cp1.wait()

        def body(t, carry):
            for cc in range(C // LANES):
                sl = pl.ds(cc * LANES, LANES)
                a_v[t, sl] = a_v[t, sl] + b_v[t, sl]
            return carry

        lax.fori_loop(0, TOKW, body, 0)
        pltpu.sync_copy(a_v, out_hbm.at[pl.ds(base, TOKW)])

    return _sc_combine


def kernel(x, Wr, br, W1, b1, W2, b2):
    x_flat = x.reshape(N, C)
    br2 = br.reshape(1, E)
    b1r = b1.reshape(E, 1, H)
    b2r = b2.reshape(E, 1, C)

    aux, w0, w1, pos0, pos1, bexp, nact, ru = _run_router(x_flat, Wr, br2)
    pos0f = pos0.reshape(N)
    pos1f = pos1.reshape(N)

    xs, ws = _get_sc_dispatch()(x_flat, w0, w1, pos0f, pos1f)
    ys = _run_ffn(bexp.reshape(G + 1), nact.reshape(1), ru.reshape(G + 1),
                  xs, ws, W1, b1r, W2, b2r)
    out = _get_sc_combine()(ys, pos0f, pos1f)

    return out.reshape(B, T, C), aux[0, 0]


# speedup vs baseline: 2.1716x; 1.0124x over previous
import functools
import math

import jax
import jax.numpy as jnp
from jax import lax
from jax.experimental import pallas as pl
from jax.experimental.pallas import tpu as pltpu
from jax.experimental.pallas import tpu_sc as plsc

B, T, C = 1, 2048, 768
E = 8
TOPK = 2
H = 4 * C
N = B * T
S = N * TOPK

BLK = 512
G = (S - (E - 1) + BLK - 1) // BLK + (E - 1)
NPAD = G * BLK

NW = 32
TOKW = N // NW
LANES = 16
WROW = 128
C2 = C // 2


def _router_body(x_ref, wr_ref, br_ref,
                 aux_ref, w0_ref, w1_ref, pos0_ref, pos1_ref,
                 bexp_ref, nact_ref, ru_ref):
    xb = x_ref[...]
    logits = jnp.dot(xb, wr_ref[...], preferred_element_type=jnp.float32)
    logits = logits + br_ref[...]
    m = jnp.max(logits, axis=1, keepdims=True)
    ex = jnp.exp(logits - m)
    gates = ex / jnp.sum(ex, axis=1, keepdims=True)

    mean_gates = jnp.sum(gates, axis=0, keepdims=True) / float(N)
    aux_ref[...] = jnp.mean((mean_gates - 1.0 / E) ** 2).reshape(1, 1)

    lane = lax.broadcasted_iota(jnp.int32, (N, E), 1)
    p0 = jnp.max(gates, axis=1, keepdims=True)
    e0 = jnp.min(jnp.where(gates == p0, lane, E), axis=1, keepdims=True)
    g2 = jnp.where(lane == e0, -jnp.inf, gates)
    p1 = jnp.max(g2, axis=1, keepdims=True)
    e1 = jnp.min(jnp.where(g2 == p1, lane, E), axis=1, keepdims=True)

    w0_ref[...] = p0 * jnp.ones((1, WROW), jnp.float32)
    w1_ref[...] = p1 * jnp.ones((1, WROW), jnp.float32)

    oh0 = (lane == e0).astype(jnp.float32)
    oh1 = (lane == e1).astype(jnp.float32)
    cat = jnp.concatenate([oh0, oh1], axis=0)

    c3 = cat.reshape(S // 8, 8, E)
    for k in (1, 2, 4):
        sh = jnp.concatenate(
            [jnp.zeros((S // 8, k, E), jnp.float32), c3[:, :8 - k, :]], axis=1)
        c3 = c3 + sh
    chunk_tot = c3[:, 7, :]
    ct = chunk_tot
    for k in (1, 2, 4, 8, 16, 32, 64, 128, 256):
        sh = jnp.concatenate(
            [jnp.zeros((k, E), jnp.float32), ct[:512 - k, :]], axis=0)
        ct = ct + sh
    excl_chunk = ct - chunk_tot
    csum = c3 + excl_chunk[:, None, :]
    csum = csum.reshape(S, E)

    counts = ct[511:512, :]
    pc = jnp.floor((counts + (BLK - 1)) / BLK) * BLK

    starts = []
    cumincl = []
    acc = jnp.zeros((1, 1), jnp.float32)
    for e in range(E):
        pce = lax.slice(pc, (0, e), (1, e + 1))
        starts.append(acc)
        acc = acc + pce
        cumincl.append(acc)
    start_row = jnp.concatenate(starts, axis=1)

    pos_all = jnp.sum(cat * (start_row + csum), axis=1, keepdims=True) - 1.0
    pos_all = pos_all.astype(jnp.int32)
    pos0_ref[...] = pos_all[:N]
    pos1_ref[...] = pos_all[N:]

    bstart = lax.broadcasted_iota(jnp.int32, (G + 1, 1), 0).astype(
        jnp.float32) * BLK
    be = jnp.zeros((G + 1, 1), jnp.float32)
    for e in range(E):
        be = be + (cumincl[e] <= bstart).astype(jnp.float32)
    be = jnp.minimum(be, E - 1)
    bexp_ref[...] = be.astype(jnp.int32)
    nact_ref[...] = (acc / BLK).astype(jnp.int32)

    ren = jnp.zeros((G + 1, 1), jnp.float32)
    for e in range(E):
        cnt_e = lax.slice(counts, (0, e), (1, e + 1))
        ren = ren + jnp.where(be == e, starts[e] + cnt_e, 0.0)
    ru_ref[...] = jnp.clip(ren - bstart, 0.0, BLK).astype(jnp.int32)


def _run_router(x_flat, Wr, br2):
    return pl.pallas_call(
        _router_body,
        in_specs=[
            pl.BlockSpec((N, C), lambda: (0, 0)),
            pl.BlockSpec((C, E), lambda: (0, 0)),
            pl.BlockSpec((1, E), lambda: (0, 0)),
        ],
        out_specs=[
            pl.BlockSpec((1, 1), lambda: (0, 0)),
            pl.BlockSpec((N, WROW), lambda: (0, 0)),
            pl.BlockSpec((N, WROW), lambda: (0, 0)),
            pl.BlockSpec((N, 1), lambda: (0, 0)),
            pl.BlockSpec((N, 1), lambda: (0, 0)),
            pl.BlockSpec((G + 1, 1), lambda: (0, 0)),
            pl.BlockSpec((1, 1), lambda: (0, 0)),
            pl.BlockSpec((G + 1, 1), lambda: (0, 0)),
        ],
        out_shape=[
            jax.ShapeDtypeStruct((1, 1), jnp.float32),
            jax.ShapeDtypeStruct((N, WROW), jnp.float32),
            jax.ShapeDtypeStruct((N, WROW), jnp.float32),
            jax.ShapeDtypeStruct((N, 1), jnp.int32),
            jax.ShapeDtypeStruct((N, 1), jnp.int32),
            jax.ShapeDtypeStruct((G + 1, 1), jnp.int32),
            jax.ShapeDtypeStruct((1, 1), jnp.int32),
            jax.ShapeDtypeStruct((G + 1, 1), jnp.int32),
        ],
    )(x_flat, Wr, br2)


@functools.cache
def _get_sc_dispatch():
    mesh = plsc.VectorSubcoreMesh(core_axis_name="c", subcore_axis_name="s")

    @functools.partial(
        pl.kernel,
        mesh=mesh,
        out_type=[
            jax.ShapeDtypeStruct((NPAD, C), jnp.float32),
            jax.ShapeDtypeStruct((NPAD, WROW), jnp.float32),
        ],
        scratch_types=[
            pltpu.VMEM((TOKW,), jnp.int32),
            pltpu.VMEM((TOKW,), jnp.int32),
            pltpu.VMEM((TOKW, C), jnp.float32),
            pltpu.VMEM((TOKW, WROW), jnp.float32),
            pltpu.VMEM((TOKW, WROW), jnp.float32),
            pltpu.SemaphoreType.DMA,
            pltpu.SemaphoreType.DMA,
        ],
    )
    def _sc_dispatch(x_hbm, w0_hbm, w1_hbm, pos0_hbm, pos1_hbm,
                     out_hbm, ws_hbm, idx0_v, idx1_v, rows_v, w0_v, w1_v,
                     sem, sem2):
        wid = lax.axis_index("s") * 2 + lax.axis_index("c")
        base = wid * TOKW
        ld0 = pltpu.async_copy(pos0_hbm.at[pl.ds(base, TOKW)], idx0_v, sem2)
        ld1 = pltpu.async_copy(pos1_hbm.at[pl.ds(base, TOKW)], idx1_v, sem2)
        ld2 = pltpu.async_copy(x_hbm.at[pl.ds(base, TOKW)], rows_v, sem2)
        ld3 = pltpu.async_copy(w0_hbm.at[pl.ds(base, TOKW)], w0_v, sem2)
        ld4 = pltpu.async_copy(w1_hbm.at[pl.ds(base, TOKW)], w1_v, sem2)
        ld0.wait()
        ld1.wait()
        ld2.wait()
        ld3.wait()
        ld4.wait()
        cpa = pltpu.async_copy(rows_v, out_hbm.at[idx0_v], sem)
        cpb = pltpu.async_copy(rows_v, out_hbm.at[idx1_v], sem)
        cpc = pltpu.async_copy(w0_v, ws_hbm.at[idx0_v], sem)
        cpd = pltpu.async_copy(w1_v, ws_hbm.at[idx1_v], sem)
        cpa.wait()
        cpb.wait()
        cpc.wait()
        cpd.wait()

    return _sc_dispatch


SUB = 128
NSUB = BLK // SUB


def _ffn_body(bexp_ref, nact_ref, ru_ref, xs_ref, ws_ref, w1_ref, b1_ref,
              w2_ref, b2_ref, o_ref):
    b = pl.program_id(0)

    @pl.when(b < nact_ref[0])
    def _():
        rb = ru_ref[b]

        def make_path(rows):
            def path():
                h = jnp.dot(xs_ref[:rows, :], w1_ref[0],
                            preferred_element_type=jnp.float32)
                h = h + b1_ref[0]
                h = 0.5 * h * (1.0 + lax.erf(h * (1.0 / math.sqrt(2.0))))
                y = jnp.dot(h, w2_ref[0], preferred_element_type=jnp.float32)
                o_ref[:rows, :] = (y + b2_ref[0]) * ws_ref[:rows, 0:1]
            return path

        for sub in range(1, NSUB + 1):
            cond = rb > (sub - 1) * SUB
            if sub < NSUB:
                cond = jnp.logical_and(cond, rb <= sub * SUB)
            pl.when(cond)(make_path(sub * SUB))


def _run_ffn(bexp, nact, ru, xs, ws, W1, b1r, W2, b2r):
    grid_spec = pltpu.PrefetchScalarGridSpec(
        num_scalar_prefetch=3,
        grid=(G,),
        in_specs=[
            pl.BlockSpec((BLK, C), lambda b, be, na, ru: (b, 0)),
            pl.BlockSpec((BLK, WROW), lambda b, be, na, ru: (b, 0)),
            pl.BlockSpec((1, C, H), lambda b, be, na, ru: (be[b], 0, 0)),
            pl.BlockSpec((1, 1, H), lambda b, be, na, ru: (be[b], 0, 0)),
            pl.BlockSpec((1, H, C), lambda b, be, na, ru: (be[b], 0, 0)),
            pl.BlockSpec((1, 1, C), lambda b, be, na, ru: (be[b], 0, 0)),
        ],
        out_specs=pl.BlockSpec((BLK, C), lambda b, be, na, ru: (b, 0)),
    )
    return pl.pallas_call(
        _ffn_body,
        grid_spec=grid_spec,
        out_shape=jax.ShapeDtypeStruct((NPAD, C), jnp.float32),
    )(bexp, nact, ru, xs, ws, W1, b1r, W2, b2r)


@functools.cache
def _get_sc_combine():
    mesh = plsc.VectorSubcoreMesh(core_axis_name="c", subcore_axis_name="s")

    @functools.partial(
        pl.kernel,
        mesh=mesh,
        out_type=jax.ShapeDtypeStruct((N, C), jnp.float32),
        scratch_types=[
            pltpu.VMEM((TOKW,), jnp.int32),
            pltpu.VMEM((TOKW,), jnp.int32),
            pltpu.VMEM((TOKW, C), jnp.float32),
            pltpu.VMEM((TOKW, C), jnp.float32),
            pltpu.SemaphoreType.DMA,
            pltpu.SemaphoreType.DMA,
        ],
    )
    def _sc_combine(y_hbm, pos0_hbm, pos1_hbm, out_hbm,
                    idx0_v, idx1_v, a_v, b_v, sem, sem2):
        wid = lax.axis_index("s") * 2 + lax.axis_index("c")
        base = wid * TOKW
        ld0 = pltpu.async_copy(pos0_hbm.at[pl.ds(base, TOKW)], idx0_v, sem2)
        ld1 = pltpu.async_copy(pos1_hbm.at[pl.ds(base, TOKW)], idx1_v, sem2)
        ld0.wait()
        ld1.wait()
        cp0 = pltpu.async_copy(y_hbm.at[idx0_v], a_v, sem)
        cp1 = pltpu.async_copy(y_hbm.at[idx1_v], b_v, sem)
        cp0.wait()
        cp1.wait()

        def body(t, carry):
            for cc in range(C // LANES):
                sl = pl.ds(cc * LANES, LANES)
                a_v[t, sl] = a_v[t, sl] + b_v[t, sl]
            return carry

        lax.fori_loop(0, TOKW, body, 0)
        pltpu.sync_copy(a_v, out_hbm.at[pl.ds(base, TOKW)])

    return _sc_combine


def kernel(x, Wr, br, W1, b1, W2, b2):
    x_flat = x.reshape(N, C)
    br2 = br.reshape(1, E)
    b1r = b1.reshape(E, 1, H)
    b2r = b2.reshape(E, 1, C)

    aux, w0, w1, pos0, pos1, bexp, nact, ru = _run_router(x_flat, Wr, br2)
    pos0f = pos0.reshape(N)
    pos1f = pos1.reshape(N)

    xs, ws = _get_sc_dispatch()(x_flat, w0, w1, pos0f, pos1f)
    ys = _run_ffn(bexp.reshape(G + 1), nact.reshape(1), ru.reshape(G + 1),
                  xs, ws, W1, b1r, W2, b2r)
    out = _get_sc_combine()(ys, pos0f, pos1f)

    return out.reshape(B, T, C), aux[0, 0]


# speedup vs baseline: 2.1868x; 1.0070x over previous
import functools
import math

import jax
import jax.numpy as jnp
from jax import lax
from jax.experimental import pallas as pl
from jax.experimental.pallas import tpu as pltpu
from jax.experimental.pallas import tpu_sc as plsc

B, T, C = 1, 2048, 768
E = 8
TOPK = 2
H = 4 * C
N = B * T
S = N * TOPK

BLK = 512
G = (S - (E - 1) + BLK - 1) // BLK + (E - 1)
NPAD = G * BLK

NW = 32
TOKW = N // NW
LANES = 16
WROW = 128
C2 = C // 2


def _router_body(x_ref, wr_ref, br_ref,
                 aux_ref, w0_ref, w1_ref, pos0_ref, pos1_ref,
                 bexp_ref, nact_ref, ru_ref):
    xb = x_ref[...]
    logits = jnp.dot(xb, wr_ref[...], preferred_element_type=jnp.float32)
    logits = logits + br_ref[...]
    m = jnp.max(logits, axis=1, keepdims=True)
    ex = jnp.exp(logits - m)
    gates = ex / jnp.sum(ex, axis=1, keepdims=True)

    mean_gates = jnp.sum(gates, axis=0, keepdims=True) / float(N)
    aux_ref[...] = jnp.mean((mean_gates - 1.0 / E) ** 2).reshape(1, 1)

    lane = lax.broadcasted_iota(jnp.int32, (N, E), 1)
    p0 = jnp.max(gates, axis=1, keepdims=True)
    e0 = jnp.min(jnp.where(gates == p0, lane, E), axis=1, keepdims=True)
    g2 = jnp.where(lane == e0, -jnp.inf, gates)
    p1 = jnp.max(g2, axis=1, keepdims=True)
    e1 = jnp.min(jnp.where(g2 == p1, lane, E), axis=1, keepdims=True)

    w0_ref[...] = p0 * jnp.ones((1, WROW), jnp.float32)
    w1_ref[...] = p1 * jnp.ones((1, WROW), jnp.float32)

    oh0 = (lane == e0).astype(jnp.float32)
    oh1 = (lane == e1).astype(jnp.float32)
    cat = jnp.concatenate([oh0, oh1], axis=0)

    c3 = cat.reshape(S // 8, 8, E)
    for k in (1, 2, 4):
        sh = jnp.concatenate(
            [jnp.zeros((S // 8, k, E), jnp.float32), c3[:, :8 - k, :]], axis=1)
        c3 = c3 + sh
    chunk_tot = c3[:, 7, :]
    g3 = chunk_tot.reshape(64, 8, E)
    for k in (1, 2, 4):
        sh = jnp.concatenate(
            [jnp.zeros((64, k, E), jnp.float32), g3[:, :8 - k, :]], axis=1)
        g3 = g3 + sh
    gt = g3[:, 7, :]
    for k in (1, 2, 4, 8, 16, 32):
        sh = jnp.concatenate(
            [jnp.zeros((k, E), jnp.float32), gt[:64 - k, :]], axis=0)
        gt = gt + sh
    excl_group = gt - g3[:, 7:8, :].reshape(64, E)
    ct = (g3 + excl_group[:, None, :]).reshape(512, E)
    excl_chunk = ct - chunk_tot
    csum = c3 + excl_chunk[:, None, :]
    csum = csum.reshape(S, E)

    counts = ct[511:512, :]
    pc = jnp.floor((counts + (BLK - 1)) / BLK) * BLK

    starts = []
    cumincl = []
    acc = jnp.zeros((1, 1), jnp.float32)
    for e in range(E):
        pce = lax.slice(pc, (0, e), (1, e + 1))
        starts.append(acc)
        acc = acc + pce
        cumincl.append(acc)
    start_row = jnp.concatenate(starts, axis=1)

    pos_all = jnp.sum(cat * (start_row + csum), axis=1, keepdims=True) - 1.0
    pos_all = pos_all.astype(jnp.int32)
    pos0_ref[...] = pos_all[:N]
    pos1_ref[...] = pos_all[N:]

    bstart = lax.broadcasted_iota(jnp.int32, (G + 1, 1), 0).astype(
        jnp.float32) * BLK
    be = jnp.zeros((G + 1, 1), jnp.float32)
    for e in range(E):
        be = be + (cumincl[e] <= bstart).astype(jnp.float32)
    be = jnp.minimum(be, E - 1)
    bexp_ref[...] = be.astype(jnp.int32)
    nact_ref[...] = (acc / BLK).astype(jnp.int32)

    ren = jnp.zeros((G + 1, 1), jnp.float32)
    for e in range(E):
        cnt_e = lax.slice(counts, (0, e), (1, e + 1))
        ren = ren + jnp.where(be == e, starts[e] + cnt_e, 0.0)
    ru_ref[...] = jnp.clip(ren - bstart, 0.0, BLK).astype(jnp.int32)


def _run_router(x_flat, Wr, br2):
    return pl.pallas_call(
        _router_body,
        in_specs=[
            pl.BlockSpec((N, C), lambda: (0, 0)),
            pl.BlockSpec((C, E), lambda: (0, 0)),
            pl.BlockSpec((1, E), lambda: (0, 0)),
        ],
        out_specs=[
            pl.BlockSpec((1, 1), lambda: (0, 0)),
            pl.BlockSpec((N, WROW), lambda: (0, 0)),
            pl.BlockSpec((N, WROW), lambda: (0, 0)),
            pl.BlockSpec((N, 1), lambda: (0, 0)),
            pl.BlockSpec((N, 1), lambda: (0, 0)),
            pl.BlockSpec((G + 1, 1), lambda: (0, 0)),
            pl.BlockSpec((1, 1), lambda: (0, 0)),
            pl.BlockSpec((G + 1, 1), lambda: (0, 0)),
        ],
        out_shape=[
            jax.ShapeDtypeStruct((1, 1), jnp.float32),
            jax.ShapeDtypeStruct((N, WROW), jnp.float32),
            jax.ShapeDtypeStruct((N, WROW), jnp.float32),
            jax.ShapeDtypeStruct((N, 1), jnp.int32),
            jax.ShapeDtypeStruct((N, 1), jnp.int32),
            jax.ShapeDtypeStruct((G + 1, 1), jnp.int32),
            jax.ShapeDtypeStruct((1, 1), jnp.int32),
            jax.ShapeDtypeStruct((G + 1, 1), jnp.int32),
        ],
    )(x_flat, Wr, br2)


@functools.cache
def _get_sc_dispatch():
    mesh = plsc.VectorSubcoreMesh(core_axis_name="c", subcore_axis_name="s")

    @functools.partial(
        pl.kernel,
        mesh=mesh,
        out_type=[
            jax.ShapeDtypeStruct((NPAD, C), jnp.float32),
            jax.ShapeDtypeStruct((NPAD, WROW), jnp.float32),
        ],
        scratch_types=[
            pltpu.VMEM((TOKW,), jnp.int32),
            pltpu.VMEM((TOKW,), jnp.int32),
            pltpu.VMEM((TOKW, C), jnp.float32),
            pltpu.VMEM((TOKW, WROW), jnp.float32),
            pltpu.VMEM((TOKW, WROW), jnp.float32),
            pltpu.SemaphoreType.DMA,
            pltpu.SemaphoreType.DMA,
        ],
    )
    def _sc_dispatch(x_hbm, w0_hbm, w1_hbm, pos0_hbm, pos1_hbm,
                     out_hbm, ws_hbm, idx0_v, idx1_v, rows_v, w0_v, w1_v,
                     sem, sem2):
        wid = lax.axis_index("s") * 2 + lax.axis_index("c")
        base = wid * TOKW
        ld0 = pltpu.async_copy(pos0_hbm.at[pl.ds(base, TOKW)], idx0_v, sem2)
        ld1 = pltpu.async_copy(pos1_hbm.at[pl.ds(base, TOKW)], idx1_v, sem2)
        ld2 = pltpu.async_copy(x_hbm.at[pl.ds(base, TOKW)], rows_v, sem2)
        ld3 = pltpu.async_copy(w0_hbm.at[pl.ds(base, TOKW)], w0_v, sem2)
        ld4 = pltpu.async_copy(w1_hbm.at[pl.ds(base, TOKW)], w1_v, sem2)
        ld0.wait()
        ld1.wait()
        ld2.wait()
        ld3.wait()
        ld4.wait()
        cpa = pltpu.async_copy(rows_v, out_hbm.at[idx0_v], sem)
        cpb = pltpu.async_copy(rows_v, out_hbm.at[idx1_v], sem)
        cpc = pltpu.async_copy(w0_v, ws_hbm.at[idx0_v], sem)
        cpd = pltpu.async_copy(w1_v, ws_hbm.at[idx1_v], sem)
        cpa.wait()
        cpb.wait()
        cpc.wait()
        cpd.wait()

    return _sc_dispatch


SUB = 128
NSUB = BLK // SUB


def _ffn_body(bexp_ref, nact_ref, ru_ref, xs_ref, ws_ref, w1_ref, b1_ref,
              w2_ref, b2_ref, o_ref):
    b = pl.program_id(0)

    @pl.when(b < nact_ref[0])
    def _():
        rb = ru_ref[b]

        def make_path(rows):
            def path():
                h = jnp.dot(xs_ref[:rows, :], w1_ref[0],
                            preferred_element_type=jnp.float32)
                h = h + b1_ref[0]
                h = 0.5 * h * (1.0 + lax.erf(h * (1.0 / math.sqrt(2.0))))
                y = jnp.dot(h, w2_ref[0], preferred_element_type=jnp.float32)
                o_ref[:rows, :] = (y + b2_ref[0]) * ws_ref[:rows, 0:1]
            return path

        for sub in range(1, NSUB + 1):
            cond = rb > (sub - 1) * SUB
            if sub < NSUB:
                cond = jnp.logical_and(cond, rb <= sub * SUB)
            pl.when(cond)(make_path(sub * SUB))


def _run_ffn(bexp, nact, ru, xs, ws, W1, b1r, W2, b2r):
    grid_spec = pltpu.PrefetchScalarGridSpec(
        num_scalar_prefetch=3,
        grid=(G,),
        in_specs=[
            pl.BlockSpec((BLK, C), lambda b, be, na, ru: (b, 0)),
            pl.BlockSpec((BLK, WROW), lambda b, be, na, ru: (b, 0)),
            pl.BlockSpec((1, C, H), lambda b, be, na, ru: (be[b], 0, 0)),
            pl.BlockSpec((1, 1, H), lambda b, be, na, ru: (be[b], 0, 0)),
            pl.BlockSpec((1, H, C), lambda b, be, na, ru: (be[b], 0, 0)),
            pl.BlockSpec((1, 1, C), lambda b, be, na, ru: (be[b], 0, 0)),
        ],
        out_specs=pl.BlockSpec((BLK, C), lambda b, be, na, ru: (b, 0)),
    )
    return pl.pallas_call(
        _ffn_body,
        grid_spec=grid_spec,
        out_shape=jax.ShapeDtypeStruct((NPAD, C), jnp.float32),
    )(bexp, nact, ru, xs, ws, W1, b1r, W2, b2r)


@functools.cache
def _get_sc_combine():
    mesh = plsc.VectorSubcoreMesh(core_axis_name="c", subcore_axis_name="s")

    @functools.partial(
        pl.kernel,
        mesh=mesh,
        out_type=jax.ShapeDtypeStruct((N, C), jnp.float32),
        scratch_types=[
            pltpu.VMEM((TOKW,), jnp.int32),
            pltpu.VMEM((TOKW,), jnp.int32),
            pltpu.VMEM((TOKW, C), jnp.float32),
            pltpu.VMEM((TOKW, C), jnp.float32),
            pltpu.SemaphoreType.DMA,
            pltpu.SemaphoreType.DMA,
        ],
    )
    def _sc_combine(y_hbm, pos0_hbm, pos1_hbm, out_hbm,
                    idx0_v, idx1_v, a_v, b_v, sem, sem2):
        wid = lax.axis_index("s") * 2 + lax.axis_index("c")
        base = wid * TOKW
        ld0 = pltpu.async_copy(pos0_hbm.at[pl.ds(base, TOKW)], idx0_v, sem2)
        ld1 = pltpu.async_copy(pos1_hbm.at[pl.ds(base, TOKW)], idx1_v, sem2)
        ld0.wait()
        ld1.wait()
        cp0 = pltpu.async_copy(y_hbm.at[idx0_v], a_v, sem)
        cp1 = pltpu.async_copy(y_hbm.at[idx1_v], b_v, sem)
        cp0.wait()
        cp1.wait()

        def body(t, carry):
            for cc in range(C // LANES):
                sl = pl.ds(cc * LANES, LANES)
                a_v[t, sl] = a_v[t, sl] + b_v[t, sl]
            return carry

        lax.fori_loop(0, TOKW, body, 0)
        pltpu.sync_copy(a_v, out_hbm.at[pl.ds(base, TOKW)])

    return _sc_combine


def kernel(x, Wr, br, W1, b1, W2, b2):
    x_flat = x.reshape(N, C)
    br2 = br.reshape(1, E)
    b1r = b1.reshape(E, 1, H)
    b2r = b2.reshape(E, 1, C)

    aux, w0, w1, pos0, pos1, bexp, nact, ru = _run_router(x_flat, Wr, br2)
    pos0f = pos0.reshape(N)
    pos1f = pos1.reshape(N)

    xs, ws = _get_sc_dispatch()(x_flat, w0, w1, pos0f, pos1f)
    ys = _run_ffn(bexp.reshape(G + 1), nact.reshape(1), ru.reshape(G + 1),
                  xs, ws, W1, b1r, W2, b2r)
    out = _get_sc_combine()(ys, pos0f, pos1f)

    return out.reshape(B, T, C), aux[0, 0]


# speedup vs baseline: 2.1961x; 1.0042x over previous
import functools
import math

import jax
import jax.numpy as jnp
from jax import lax
from jax.experimental import pallas as pl
from jax.experimental.pallas import tpu as pltpu
from jax.experimental.pallas import tpu_sc as plsc

B, T, C = 1, 2048, 768
E = 8
TOPK = 2
H = 4 * C
N = B * T
S = N * TOPK

BLK = 512
G = (S - (E - 1) + BLK - 1) // BLK + (E - 1)
NPAD = G * BLK

NW = 32
TOKW = N // NW
LANES = 16
WROW = 128
C2 = C // 2


def _router_body(x_ref, wr_ref, br_ref,
                 aux_ref, w0_ref, w1_ref, pos0_ref, pos1_ref,
                 bexp_ref, nact_ref, ru_ref):
    xb = x_ref[...]
    logits = jnp.dot(xb, wr_ref[...], preferred_element_type=jnp.float32)
    logits = logits + br_ref[...]
    m = jnp.max(logits, axis=1, keepdims=True)
    ex = jnp.exp(logits - m)
    gates = ex / jnp.sum(ex, axis=1, keepdims=True)

    mean_gates = jnp.sum(gates, axis=0, keepdims=True) / float(N)
    aux_ref[...] = jnp.mean((mean_gates - 1.0 / E) ** 2).reshape(1, 1)

    lane = lax.broadcasted_iota(jnp.int32, (N, E), 1)
    p0 = jnp.max(gates, axis=1, keepdims=True)
    e0 = jnp.min(jnp.where(gates == p0, lane, E), axis=1, keepdims=True)
    g2 = jnp.where(lane == e0, -jnp.inf, gates)
    p1 = jnp.max(g2, axis=1, keepdims=True)
    e1 = jnp.min(jnp.where(g2 == p1, lane, E), axis=1, keepdims=True)

    w0_ref[...] = p0 * jnp.ones((1, WROW), jnp.float32)
    w1_ref[...] = p1 * jnp.ones((1, WROW), jnp.float32)

    oh0 = (lane == e0).astype(jnp.float32)
    oh1 = (lane == e1).astype(jnp.float32)
    cat = jnp.concatenate([oh0, oh1], axis=0)

    c3 = cat.reshape(S // 8, 8, E)
    for k in (1, 2, 4):
        sh = jnp.concatenate(
            [jnp.zeros((S // 8, k, E), jnp.float32), c3[:, :8 - k, :]], axis=1)
        c3 = c3 + sh
    chunk_tot = c3[:, 7, :]
    g3 = chunk_tot.reshape(64, 8, E)
    for k in (1, 2, 4):
        sh = jnp.concatenate(
            [jnp.zeros((64, k, E), jnp.float32), g3[:, :8 - k, :]], axis=1)
        g3 = g3 + sh
    gt = g3[:, 7, :]
    for k in (1, 2, 4, 8, 16, 32):
        sh = jnp.concatenate(
            [jnp.zeros((k, E), jnp.float32), gt[:64 - k, :]], axis=0)
        gt = gt + sh
    excl_group = gt - g3[:, 7:8, :].reshape(64, E)
    ct = (g3 + excl_group[:, None, :]).reshape(512, E)
    excl_chunk = ct - chunk_tot
    csum = c3 + excl_chunk[:, None, :]
    csum = csum.reshape(S, E)

    counts = ct[511:512, :]
    pc = jnp.floor((counts + (BLK - 1)) / BLK) * BLK

    starts = []
    cumincl = []
    acc = jnp.zeros((1, 1), jnp.float32)
    for e in range(E):
        pce = lax.slice(pc, (0, e), (1, e + 1))
        starts.append(acc)
        acc = acc + pce
        cumincl.append(acc)
    start_row = jnp.concatenate(starts, axis=1)

    pos_all = jnp.sum(cat * (start_row + csum), axis=1, keepdims=True) - 1.0
    pos_all = pos_all.astype(jnp.int32)
    pos0_ref[...] = pos_all[:N]
    pos1_ref[...] = pos_all[N:]

    bstart = lax.broadcasted_iota(jnp.int32, (G + 1, 1), 0).astype(
        jnp.float32) * BLK
    be = jnp.zeros((G + 1, 1), jnp.float32)
    for e in range(E):
        be = be + (cumincl[e] <= bstart).astype(jnp.float32)
    be = jnp.minimum(be, E - 1)
    bexp_ref[...] = be.astype(jnp.int32)
    nact_ref[...] = (acc / BLK).astype(jnp.int32)

    ren = jnp.zeros((G + 1, 1), jnp.float32)
    for e in range(E):
        cnt_e = lax.slice(counts, (0, e), (1, e + 1))
        ren = ren + jnp.where(be == e, starts[e] + cnt_e, 0.0)
    ru_ref[...] = jnp.clip(ren - bstart, 0.0, BLK).astype(jnp.int32)


def _run_router(x_flat, Wr, br2):
    return pl.pallas_call(
        _router_body,
        in_specs=[
            pl.BlockSpec((N, C), lambda: (0, 0)),
            pl.BlockSpec((C, E), lambda: (0, 0)),
            pl.BlockSpec((1, E), lambda: (0, 0)),
        ],
        out_specs=[
            pl.BlockSpec((1, 1), lambda: (0, 0)),
            pl.BlockSpec((N, WROW), lambda: (0, 0)),
            pl.BlockSpec((N, WROW), lambda: (0, 0)),
            pl.BlockSpec((N, 1), lambda: (0, 0)),
            pl.BlockSpec((N, 1), lambda: (0, 0)),
            pl.BlockSpec((G + 1, 1), lambda: (0, 0)),
            pl.BlockSpec((1, 1), lambda: (0, 0)),
            pl.BlockSpec((G + 1, 1), lambda: (0, 0)),
        ],
        out_shape=[
            jax.ShapeDtypeStruct((1, 1), jnp.float32),
            jax.ShapeDtypeStruct((N, WROW), jnp.float32),
            jax.ShapeDtypeStruct((N, WROW), jnp.float32),
            jax.ShapeDtypeStruct((N, 1), jnp.int32),
            jax.ShapeDtypeStruct((N, 1), jnp.int32),
            jax.ShapeDtypeStruct((G + 1, 1), jnp.int32),
            jax.ShapeDtypeStruct((1, 1), jnp.int32),
            jax.ShapeDtypeStruct((G + 1, 1), jnp.int32),
        ],
    )(x_flat, Wr, br2)


@functools.cache
def _get_sc_dispatch():
    mesh = plsc.VectorSubcoreMesh(core_axis_name="c", subcore_axis_name="s")

    @functools.partial(
        pl.kernel,
        mesh=mesh,
        out_type=[
            jax.ShapeDtypeStruct((NPAD, C), jnp.float32),
            jax.ShapeDtypeStruct((NPAD, WROW), jnp.float32),
        ],
        scratch_types=[
            pltpu.VMEM((TOKW,), jnp.int32),
            pltpu.VMEM((TOKW,), jnp.int32),
            pltpu.VMEM((TOKW, C), jnp.float32),
            pltpu.VMEM((TOKW, WROW), jnp.float32),
            pltpu.VMEM((TOKW, WROW), jnp.float32),
            pltpu.SemaphoreType.DMA,
            pltpu.SemaphoreType.DMA,
        ],
    )
    def _sc_dispatch(x_hbm, w0_hbm, w1_hbm, pos0_hbm, pos1_hbm,
                     out_hbm, ws_hbm, idx0_v, idx1_v, rows_v, w0_v, w1_v,
                     sem, sem2):
        wid = lax.axis_index("s") * 2 + lax.axis_index("c")
        base = wid * TOKW
        ld0 = pltpu.async_copy(pos0_hbm.at[pl.ds(base, TOKW)], idx0_v, sem2)
        ld1 = pltpu.async_copy(pos1_hbm.at[pl.ds(base, TOKW)], idx1_v, sem2)
        ld2 = pltpu.async_copy(x_hbm.at[pl.ds(base, TOKW)], rows_v, sem2)
        ld3 = pltpu.async_copy(w0_hbm.at[pl.ds(base, TOKW)], w0_v, sem2)
        ld4 = pltpu.async_copy(w1_hbm.at[pl.ds(base, TOKW)], w1_v, sem2)
        ld0.wait()
        ld1.wait()
        ld2.wait()
        ld3.wait()
        ld4.wait()
        cpa = pltpu.async_copy(rows_v, out_hbm.at[idx0_v], sem)
        cpb = pltpu.async_copy(rows_v, out_hbm.at[idx1_v], sem)
        cpc = pltpu.async_copy(w0_v, ws_hbm.at[idx0_v], sem)
        cpd = pltpu.async_copy(w1_v, ws_hbm.at[idx1_v], sem)
        cpa.wait()
        cpb.wait()
        cpc.wait()
        cpd.wait()

    return _sc_dispatch


SUB = 128
NSUB = BLK // SUB


def _ffn_body(bexp_ref, nact_ref, ru_ref, xs_ref, ws_ref, w1_ref, b1_ref,
              w2_ref, b2_ref, o_ref):
    b = pl.program_id(0)

    @pl.when(b < nact_ref[0])
    def _():
        rb = ru_ref[b]

        def make_path(rows):
            def path():
                h = jnp.dot(xs_ref[:rows, :], w1_ref[0],
                            preferred_element_type=jnp.float32)
                h = h + b1_ref[0]
                h = 0.5 * h * (1.0 + lax.erf(h * (1.0 / math.sqrt(2.0))))
                y = jnp.dot(h, w2_ref[0], preferred_element_type=jnp.float32)
                o_ref[:rows, :] = (y + b2_ref[0]) * ws_ref[:rows, 0:1]
            return path

        for sub in range(1, NSUB + 1):
            cond = rb > (sub - 1) * SUB
            if sub < NSUB:
                cond = jnp.logical_and(cond, rb <= sub * SUB)
            pl.when(cond)(make_path(sub * SUB))


def _run_ffn(bexp, nact, ru, xs, ws, W1, b1r, W2, b2r):
    grid_spec = pltpu.PrefetchScalarGridSpec(
        num_scalar_prefetch=3,
        grid=(G,),
        in_specs=[
            pl.BlockSpec((BLK, C), lambda b, be, na, ru: (b, 0)),
            pl.BlockSpec((BLK, WROW), lambda b, be, na, ru: (b, 0)),
            pl.BlockSpec((1, C, H), lambda b, be, na, ru: (be[b], 0, 0)),
            pl.BlockSpec((1, 1, H), lambda b, be, na, ru: (be[b], 0, 0)),
            pl.BlockSpec((1, H, C), lambda b, be, na, ru: (be[b], 0, 0)),
            pl.BlockSpec((1, 1, C), lambda b, be, na, ru: (be[b], 0, 0)),
        ],
        out_specs=pl.BlockSpec((BLK, C), lambda b, be, na, ru: (b, 0)),
    )
    return pl.pallas_call(
        _ffn_body,
        grid_spec=grid_spec,
        out_shape=jax.ShapeDtypeStruct((NPAD, C), jnp.float32),
    )(bexp, nact, ru, xs, ws, W1, b1r, W2, b2r)


@functools.cache
def _get_sc_combine():
    mesh = plsc.VectorSubcoreMesh(core_axis_name="c", subcore_axis_name="s")

    @functools.partial(
        pl.kernel,
        mesh=mesh,
        out_type=jax.ShapeDtypeStruct((N, C), jnp.float32),
        scratch_types=[
            pltpu.VMEM((TOKW // 2,), jnp.int32),
            pltpu.VMEM((TOKW // 2,), jnp.int32),
            pltpu.VMEM((TOKW // 2,), jnp.int32),
            pltpu.VMEM((TOKW // 2,), jnp.int32),
            pltpu.VMEM((TOKW // 2, C), jnp.float32),
            pltpu.VMEM((TOKW // 2, C), jnp.float32),
            pltpu.VMEM((TOKW // 2, C), jnp.float32),
            pltpu.VMEM((TOKW // 2, C), jnp.float32),
            pltpu.SemaphoreType.DMA,
            pltpu.SemaphoreType.DMA,
            pltpu.SemaphoreType.DMA,
        ],
    )
    def _sc_combine(y_hbm, pos0_hbm, pos1_hbm, out_hbm,
                    i0a_v, i1a_v, i0b_v, i1b_v, a0_v, b0_v, a1_v, b1_v,
                    semA, semB, sem2):
        wid = lax.axis_index("s") * 2 + lax.axis_index("c")
        base = wid * TOKW
        half = TOKW // 2
        ld0 = pltpu.async_copy(pos0_hbm.at[pl.ds(base, half)], i0a_v, sem2)
        ld1 = pltpu.async_copy(pos1_hbm.at[pl.ds(base, half)], i1a_v, sem2)
        ld2 = pltpu.async_copy(pos0_hbm.at[pl.ds(base + half, half)], i0b_v,
                               sem2)
        ld3 = pltpu.async_copy(pos1_hbm.at[pl.ds(base + half, half)], i1b_v,
                               sem2)
        ld0.wait()
        ld1.wait()
        cpa0 = pltpu.async_copy(y_hbm.at[i0a_v], a0_v, semA)
        cpa1 = pltpu.async_copy(y_hbm.at[i1a_v], b0_v, semA)
        ld2.wait()
        ld3.wait()
        cpb0 = pltpu.async_copy(y_hbm.at[i0b_v], a1_v, semB)
        cpb1 = pltpu.async_copy(y_hbm.at[i1b_v], b1_v, semB)
        cpa0.wait()
        cpa1.wait()

        def body0(t, carry):
            for cc in range(C // LANES):
                sl = pl.ds(cc * LANES, LANES)
                a0_v[t, sl] = a0_v[t, sl] + b0_v[t, sl]
            return carry

        lax.fori_loop(0, half, body0, 0)
        st0 = pltpu.async_copy(a0_v, out_hbm.at[pl.ds(base, half)], sem2)
        cpb0.wait()
        cpb1.wait()

        def body1(t, carry):
            for cc in range(C // LANES):
                sl = pl.ds(cc * LANES, LANES)
                a1_v[t, sl] = a1_v[t, sl] + b1_v[t, sl]
            return carry

        lax.fori_loop(0, half, body1, 0)
        st0.wait()
        pltpu.sync_copy(a1_v, out_hbm.at[pl.ds(base + half, half)])

    return _sc_combine


def kernel(x, Wr, br, W1, b1, W2, b2):
    x_flat = x.reshape(N, C)
    br2 = br.reshape(1, E)
    b1r = b1.reshape(E, 1, H)
    b2r = b2.reshape(E, 1, C)

    aux, w0, w1, pos0, pos1, bexp, nact, ru = _run_router(x_flat, Wr, br2)
    pos0f = pos0.reshape(N)
    pos1f = pos1.reshape(N)

    xs, ws = _get_sc_dispatch()(x_flat, w0, w1, pos0f, pos1f)
    ys = _run_ffn(bexp.reshape(G + 1), nact.reshape(1), ru.reshape(G + 1),
                  xs, ws, W1, b1r, W2, b2r)
    out = _get_sc_combine()(ys, pos0f, pos1f)

    return out.reshape(B, T, C), aux[0, 0]
